# baseline (speedup 1.0000x reference)
import functools

import jax
import jax.numpy as jnp
from jax import lax
from jax.experimental import pallas as pl
from jax.experimental.pallas import tpu as pltpu
from jax.experimental.pallas import tpu_sc as plsc

N = 10000
E = 320000
G = 64
D = 128
DOUT = 256

NC, NS = 2, 16
NW = NC * NS
NPAD = 10240
EPT = E // NW
CHUNK = 80
NCHUNK = EPT // CHUNK
RPT = NPAD // NS

_mesh = plsc.VectorSubcoreMesh(core_axis_name="c", subcore_axis_name="s")


def _zero_rows(ref, nrows):
    z = jnp.zeros((16,), jnp.float32)

    @pl.loop(0, nrows)
    def _(i):
        for j in range(128 // 16):
            ref[i, pl.ds(j * 16, 16)] = z


@functools.partial(
    pl.kernel,
    out_type=jax.ShapeDtypeStruct((NC, NPAD), jnp.float32),
    mesh=_mesh,
    scratch_types=[
        pltpu.VMEM((NCHUNK, CHUNK), jnp.int32),
        pltpu.VMEM((CHUNK,), jnp.float32),
        pltpu.VMEM((RPT,), jnp.float32),
        pltpu.VMEM_SHARED((NPAD,), jnp.float32),
    ],
)
def _deg_kernel(dst_hbm, out_hbm, dst_v, ones_v, z_v, hist_sp):
    cid = lax.axis_index("c")
    sid = lax.axis_index("s")
    wid = cid * NS + sid

    pltpu.sync_copy(dst_hbm.at[wid], dst_v)
    one = jnp.ones((16,), jnp.float32)
    zero = jnp.zeros((16,), jnp.float32)
    for j in range(CHUNK // 16):
        ones_v[pl.ds(j * 16, 16)] = one

    @pl.loop(0, RPT // 16)
    def _(i):
        z_v[pl.ds(i * 16, 16)] = zero

    pltpu.sync_copy(z_v, hist_sp.at[pl.ds(sid * RPT, RPT)])
    plsc.subcore_barrier()

    @pl.loop(0, NCHUNK)
    def _(j):
        pltpu.sync_copy(ones_v, hist_sp.at[dst_v.at[j]], add=True)

    plsc.subcore_barrier()
    pltpu.sync_copy(hist_sp.at[pl.ds(sid * RPT, RPT)],
                    out_hbm.at[cid, pl.ds(sid * RPT, RPT)])


EPTP = 10080
SNCHUNK = EPTP // CHUNK
HCHUNK = SNCHUNK // 2


@functools.partial(
    pl.kernel,
    out_type=jax.ShapeDtypeStruct((NC, NPAD, D), jnp.float32),
    mesh=_mesh,
    scratch_types=[
        pltpu.VMEM((HCHUNK, CHUNK), jnp.int32),
        pltpu.VMEM((HCHUNK, CHUNK), jnp.int32),
        pltpu.VMEM((CHUNK, D), jnp.float32),
        pltpu.VMEM((CHUNK, D), jnp.float32),
        pltpu.VMEM_SHARED((NPAD, D), jnp.float32),
        pltpu.SemaphoreType.DMA,
    ],
)
def _scat_kernel(hs_hbm, src_hbm, dst_hbm, out_hbm, src_v, dst_v, rows_v,
                 rows_w, acc_sp, gsem):
    cid = lax.axis_index("c")
    sid = lax.axis_index("s")
    wid = cid * NS + sid

    _zero_rows(rows_v, CHUNK)
    for r in range(RPT // CHUNK):
        pltpu.sync_copy(rows_v, acc_sp.at[pl.ds(sid * RPT + r * CHUNK, CHUNK)])
    plsc.subcore_barrier()

    for h in range(2):
        pltpu.sync_copy(src_hbm.at[wid, h], src_v)
        pltpu.sync_copy(dst_hbm.at[wid, h], dst_v)

        @pl.loop(0, HCHUNK // 2)
        def _(m):
            j = 2 * m
            pltpu.sync_copy(hs_hbm.at[src_v.at[j]], rows_v)
            d = pltpu.async_copy(hs_hbm.at[src_v.at[j + 1]], rows_w, gsem)
            pltpu.sync_copy(rows_v, acc_sp.at[dst_v.at[j]], add=True)
            d.wait()
            pltpu.sync_copy(rows_w, acc_sp.at[dst_v.at[j + 1]], add=True)

        pltpu.sync_copy(hs_hbm.at[src_v.at[HCHUNK - 1]], rows_v)
        pltpu.sync_copy(rows_v, acc_sp.at[dst_v.at[HCHUNK - 1]], add=True)

    plsc.subcore_barrier()
    pltpu.sync_copy(acc_sp.at[pl.ds(sid * RPT, RPT)],
                    out_hbm.at[cid].at[pl.ds(sid * RPT, RPT)])


_BLK = 1000
_NBLK = N // _BLK


def _tc_prep_body(deg0, deg1, x, W, dinv_o, hs_o):
    deg = deg0[...] + deg1[...] + 1.0
    dinv = lax.rsqrt(deg)
    dinv_o[...] = dinv
    hs_o[...] = jnp.dot(x[...], W[...], preferred_element_type=jnp.float32) * dinv


_tc_prep = pl.pallas_call(
    _tc_prep_body,
    grid=(_NBLK,),
    in_specs=[
        pl.BlockSpec((_BLK, 1), lambda i: (i, 0)),
        pl.BlockSpec((_BLK, 1), lambda i: (i, 0)),
        pl.BlockSpec((_BLK, D), lambda i: (i, 0)),
        pl.BlockSpec((D, D), lambda i: (0, 0)),
    ],
    out_specs=[
        pl.BlockSpec((_BLK, 1), lambda i: (i, 0)),
        pl.BlockSpec((_BLK, D), lambda i: (i, 0)),
    ],
    out_shape=[
        jax.ShapeDtypeStruct((N, 1), jnp.float32),
        jax.ShapeDtypeStruct((N, D), jnp.float32),
    ],
)


def _tc_combine_body(p0, p1, hs, dinv, b, W, hs_o):
    h = jnp.maximum(dinv[...] * (p0[...] + p1[...] + hs[...]) + b[...], 0.0)
    hs_o[...] = jnp.dot(h, W[...], preferred_element_type=jnp.float32) * dinv[...]


_tc_combine = pl.pallas_call(
    _tc_combine_body,
    grid=(_NBLK,),
    in_specs=[
        pl.BlockSpec((_BLK, D), lambda i: (i, 0)),
        pl.BlockSpec((_BLK, D), lambda i: (i, 0)),
        pl.BlockSpec((_BLK, D), lambda i: (i, 0)),
        pl.BlockSpec((_BLK, 1), lambda i: (i, 0)),
        pl.BlockSpec((1, D), lambda i: (0, 0)),
        pl.BlockSpec((D, D), lambda i: (0, 0)),
    ],
    out_specs=pl.BlockSpec((_BLK, D), lambda i: (i, 0)),
    out_shape=jax.ShapeDtypeStruct((N, D), jnp.float32),
)


def _tc_final_body(p0, p1, hs, dinv, b, batch, Wfc, bfc, out,
                   sums_ref, counts_ref):
    i = pl.program_id(0)
    h = jnp.maximum(dinv[...] * (p0[...] + p1[...] + hs[...]) + b[...], 0.0)
    gids = lax.broadcasted_iota(jnp.int32, (1, G), 1)
    onehot = jnp.where(batch[...] == gids, 1.0, 0.0)
    dn = (((0,), (0,)), ((), ()))
    ps = lax.dot_general(onehot, h, dn, preferred_element_type=jnp.float32)
    pc = lax.dot_general(onehot, jnp.ones_like(h), dn,
                         preferred_element_type=jnp.float32)

    @pl.when(i == 0)
    def _():
        sums_ref[...] = ps
        counts_ref[...] = pc

    @pl.when(i > 0)
    def _():
        sums_ref[...] += ps
        counts_ref[...] += pc

    @pl.when(i == pl.num_programs(0) - 1)
    def _():
        pooled = sums_ref[...] / jnp.maximum(counts_ref[...], 1.0)
        out[...] = jnp.dot(pooled, Wfc[...],
                           preferred_element_type=jnp.float32) + bfc[...]


_tc_final = pl.pallas_call(
    _tc_final_body,
    grid=(_NBLK,),
    in_specs=[
        pl.BlockSpec((_BLK, D), lambda i: (i, 0)),
        pl.BlockSpec((_BLK, D), lambda i: (i, 0)),
        pl.BlockSpec((_BLK, D), lambda i: (i, 0)),
        pl.BlockSpec((_BLK, 1), lambda i: (i, 0)),
        pl.BlockSpec((1, D), lambda i: (0, 0)),
        pl.BlockSpec((_BLK, 1), lambda i: (i, 0)),
        pl.BlockSpec((D, DOUT), lambda i: (0, 0)),
        pl.BlockSpec((1, DOUT), lambda i: (0, 0)),
    ],
    out_specs=pl.BlockSpec((G, DOUT), lambda i: (0, 0)),
    out_shape=jax.ShapeDtypeStruct((G, DOUT), jnp.float32),
    scratch_shapes=[
        pltpu.VMEM((G, D), jnp.float32),
        pltpu.VMEM((G, D), jnp.float32),
    ],
)


def kernel(x, edge_index, batch, W1, b1, W2, b2, W3, b3, Wfc, bfc):
    npadw = EPTP - EPT
    fake_src = jnp.zeros((NW, npadw), jnp.int32)
    fake_dst = (N + (jnp.arange(NW, dtype=jnp.int32)[:, None] * 7
                     + jnp.arange(npadw, dtype=jnp.int32)[None, :])
                % (NPAD - N))
    src = jnp.concatenate([edge_index[0].reshape(NW, EPT), fake_src],
                          axis=1).reshape(NW, 2, HCHUNK, CHUNK)
    dst = jnp.concatenate([edge_index[1].reshape(NW, EPT), fake_dst],
                          axis=1).reshape(NW, 2, HCHUNK, CHUNK)
    dstw = edge_index[1].reshape(NW, NCHUNK, CHUNK)

    degp = _deg_kernel(dstw)
    deg0 = degp[0, :N].reshape(N, 1)
    deg1 = degp[1, :N].reshape(N, 1)
    dinv, hs = _tc_prep(deg0, deg1, x, W1)

    p = _scat_kernel(hs, src, dst)
    hs = _tc_combine(p[0, :N], p[1, :N], hs, dinv, b1.reshape(1, D), W2)
    p = _scat_kernel(hs, src, dst)
    hs = _tc_combine(p[0, :N], p[1, :N], hs, dinv, b2.reshape(1, D), W3)
    p = _scat_kernel(hs, src, dst)
    return _tc_final(p[0, :N], p[1, :N], hs, dinv, b3.reshape(1, D),
                     batch.reshape(N, 1), Wfc, bfc.reshape(1, DOUT))


# speedup vs baseline: 1.2558x; 1.2558x over previous
import functools

import jax
import jax.numpy as jnp
from jax import lax
from jax.experimental import pallas as pl
from jax.experimental.pallas import tpu as pltpu
from jax.experimental.pallas import tpu_sc as plsc

N = 10000
E = 320000
G = 64
D = 128
DOUT = 256

NC, NS = 2, 16
NW = NC * NS
NPAD = 10240
EPT = E // NW
CHUNK = 80
NCHUNK = EPT // CHUNK
RPT = NPAD // NS

_mesh = plsc.VectorSubcoreMesh(core_axis_name="c", subcore_axis_name="s")


def _zero_rows(ref, nrows):
    z = jnp.zeros((16,), jnp.float32)

    @pl.loop(0, nrows)
    def _(i):
        for j in range(128 // 16):
            ref[i, pl.ds(j * 16, 16)] = z


@functools.partial(
    pl.kernel,
    out_type=jax.ShapeDtypeStruct((NC, NPAD), jnp.float32),
    mesh=_mesh,
    scratch_types=[
        pltpu.VMEM((NCHUNK, CHUNK), jnp.int32),
        pltpu.VMEM((CHUNK,), jnp.float32),
        pltpu.VMEM((RPT,), jnp.float32),
        pltpu.VMEM_SHARED((NPAD,), jnp.float32),
    ],
)
def _deg_kernel(dst_hbm, out_hbm, dst_v, ones_v, z_v, hist_sp):
    cid = lax.axis_index("c")
    sid = lax.axis_index("s")
    wid = cid * NS + sid

    pltpu.sync_copy(dst_hbm.at[wid], dst_v)
    one = jnp.ones((16,), jnp.float32)
    zero = jnp.zeros((16,), jnp.float32)
    for j in range(CHUNK // 16):
        ones_v[pl.ds(j * 16, 16)] = one

    @pl.loop(0, RPT // 16)
    def _(i):
        z_v[pl.ds(i * 16, 16)] = zero

    pltpu.sync_copy(z_v, hist_sp.at[pl.ds(sid * RPT, RPT)])
    plsc.subcore_barrier()

    @pl.loop(0, NCHUNK)
    def _(j):
        pltpu.sync_copy(ones_v, hist_sp.at[dst_v.at[j]], add=True)

    plsc.subcore_barrier()
    pltpu.sync_copy(hist_sp.at[pl.ds(sid * RPT, RPT)],
                    out_hbm.at[cid, pl.ds(sid * RPT, RPT)])


@functools.partial(
    pl.kernel,
    out_type=jax.ShapeDtypeStruct((NC, NPAD, D), jnp.float32),
    mesh=_mesh,
    scratch_types=[
        pltpu.VMEM((NCHUNK, CHUNK), jnp.int32),
        pltpu.VMEM((NCHUNK, CHUNK), jnp.int32),
        pltpu.VMEM((CHUNK, D), jnp.float32),
        pltpu.VMEM_SHARED((NPAD, D), jnp.float32),
    ],
)
def _scat_kernel(hs_hbm, src_hbm, dst_hbm, out_hbm, src_v, dst_v, rows_v,
                 acc_sp):
    cid = lax.axis_index("c")
    sid = lax.axis_index("s")
    wid = cid * NS + sid

    pltpu.sync_copy(src_hbm.at[wid], src_v)
    pltpu.sync_copy(dst_hbm.at[wid], dst_v)

    _zero_rows(rows_v, CHUNK)
    for r in range(RPT // CHUNK):
        pltpu.sync_copy(rows_v, acc_sp.at[pl.ds(sid * RPT + r * CHUNK, CHUNK)])
    plsc.subcore_barrier()

    @pl.loop(0, NCHUNK)
    def _(j):
        pltpu.sync_copy(hs_hbm.at[src_v.at[j]], rows_v)
        pltpu.sync_copy(rows_v, acc_sp.at[dst_v.at[j]], add=True)

    plsc.subcore_barrier()
    pltpu.sync_copy(acc_sp.at[pl.ds(sid * RPT, RPT)],
                    out_hbm.at[cid].at[pl.ds(sid * RPT, RPT)])


_BLK = 1000
_NBLK = N // _BLK


def _tc_prep_body(deg0, deg1, x, W, dinv_o, hs_o):
    deg = deg0[...] + deg1[...] + 1.0
    dinv = lax.rsqrt(deg)
    dinv_o[...] = dinv
    hs_o[...] = jnp.dot(x[...], W[...], preferred_element_type=jnp.float32) * dinv


_tc_prep = pl.pallas_call(
    _tc_prep_body,
    grid=(_NBLK,),
    in_specs=[
        pl.BlockSpec((_BLK, 1), lambda i: (i, 0)),
        pl.BlockSpec((_BLK, 1), lambda i: (i, 0)),
        pl.BlockSpec((_BLK, D), lambda i: (i, 0)),
        pl.BlockSpec((D, D), lambda i: (0, 0)),
    ],
    out_specs=[
        pl.BlockSpec((_BLK, 1), lambda i: (i, 0)),
        pl.BlockSpec((_BLK, D), lambda i: (i, 0)),
    ],
    out_shape=[
        jax.ShapeDtypeStruct((N, 1), jnp.float32),
        jax.ShapeDtypeStruct((N, D), jnp.float32),
    ],
)


def _tc_combine_body(p0, p1, hs, dinv, b, W, hs_o):
    h = jnp.maximum(dinv[...] * (p0[...] + p1[...] + hs[...]) + b[...], 0.0)
    hs_o[...] = jnp.dot(h, W[...], preferred_element_type=jnp.float32) * dinv[...]


_tc_combine = pl.pallas_call(
    _tc_combine_body,
    grid=(_NBLK,),
    in_specs=[
        pl.BlockSpec((_BLK, D), lambda i: (i, 0)),
        pl.BlockSpec((_BLK, D), lambda i: (i, 0)),
        pl.BlockSpec((_BLK, D), lambda i: (i, 0)),
        pl.BlockSpec((_BLK, 1), lambda i: (i, 0)),
        pl.BlockSpec((1, D), lambda i: (0, 0)),
        pl.BlockSpec((D, D), lambda i: (0, 0)),
    ],
    out_specs=pl.BlockSpec((_BLK, D), lambda i: (i, 0)),
    out_shape=jax.ShapeDtypeStruct((N, D), jnp.float32),
)


def _tc_final_body(p0, p1, hs, dinv, b, batch, Wfc, bfc, out,
                   sums_ref, counts_ref):
    i = pl.program_id(0)
    h = jnp.maximum(dinv[...] * (p0[...] + p1[...] + hs[...]) + b[...], 0.0)
    gids = lax.broadcasted_iota(jnp.int32, (1, G), 1)
    onehot = jnp.where(batch[...] == gids, 1.0, 0.0)
    dn = (((0,), (0,)), ((), ()))
    ps = lax.dot_general(onehot, h, dn, preferred_element_type=jnp.float32)
    pc = lax.dot_general(onehot, jnp.ones_like(h), dn,
                         preferred_element_type=jnp.float32)

    @pl.when(i == 0)
    def _():
        sums_ref[...] = ps
        counts_ref[...] = pc

    @pl.when(i > 0)
    def _():
        sums_ref[...] += ps
        counts_ref[...] += pc

    @pl.when(i == pl.num_programs(0) - 1)
    def _():
        pooled = sums_ref[...] / jnp.maximum(counts_ref[...], 1.0)
        out[...] = jnp.dot(pooled, Wfc[...],
                           preferred_element_type=jnp.float32) + bfc[...]


_tc_final = pl.pallas_call(
    _tc_final_body,
    grid=(_NBLK,),
    in_specs=[
        pl.BlockSpec((_BLK, D), lambda i: (i, 0)),
        pl.BlockSpec((_BLK, D), lambda i: (i, 0)),
        pl.BlockSpec((_BLK, D), lambda i: (i, 0)),
        pl.BlockSpec((_BLK, 1), lambda i: (i, 0)),
        pl.BlockSpec((1, D), lambda i: (0, 0)),
        pl.BlockSpec((_BLK, 1), lambda i: (i, 0)),
        pl.BlockSpec((D, DOUT), lambda i: (0, 0)),
        pl.BlockSpec((1, DOUT), lambda i: (0, 0)),
    ],
    out_specs=pl.BlockSpec((G, DOUT), lambda i: (0, 0)),
    out_shape=jax.ShapeDtypeStruct((G, DOUT), jnp.float32),
    scratch_shapes=[
        pltpu.VMEM((G, D), jnp.float32),
        pltpu.VMEM((G, D), jnp.float32),
    ],
)


def kernel(x, edge_index, batch, W1, b1, W2, b2, W3, b3, Wfc, bfc):
    src = edge_index[0].reshape(NW, NCHUNK, CHUNK)
    dst = edge_index[1].reshape(NW, NCHUNK, CHUNK)

    degp = _deg_kernel(dst)
    deg0 = degp[0, :N].reshape(N, 1)
    deg1 = degp[1, :N].reshape(N, 1)
    dinv, hs = _tc_prep(deg0, deg1, x, W1)

    p = _scat_kernel(hs, src, dst)
    hs = _tc_combine(p[0, :N], p[1, :N], hs, dinv, b1.reshape(1, D), W2)
    p = _scat_kernel(hs, src, dst)
    hs = _tc_combine(p[0, :N], p[1, :N], hs, dinv, b2.reshape(1, D), W3)
    p = _scat_kernel(hs, src, dst)
    return _tc_final(p[0, :N], p[1, :N], hs, dinv, b3.reshape(1, D),
                     batch.reshape(N, 1), Wfc, bfc.reshape(1, DOUT))
